# baseline (speedup 1.0000x reference)
import functools

import jax
import jax.numpy as jnp
from jax import lax
from jax.experimental import pallas as pl
from jax.experimental.pallas import tpu as pltpu
from jax.experimental.pallas import tpu_sc as plsc

BATCH = 16384
EMBED_DIM = 64

_info = plsc.get_sparse_core_info()
_NC, _NS = _info.num_cores, _info.num_subcores
_NW = _NC * _NS
_BPW = BATCH // _NW


def _make_kernel():
    mesh = plsc.VectorSubcoreMesh(core_axis_name="c", subcore_axis_name="s")

    @functools.partial(
        pl.kernel,
        mesh=mesh,
        out_type=(
            jax.ShapeDtypeStruct((BATCH, EMBED_DIM), jnp.float32),
            jax.ShapeDtypeStruct((BATCH, EMBED_DIM), jnp.float32),
        ),
        scratch_types=[
            pltpu.VMEM((_BPW,), jnp.int32),
            pltpu.VMEM((_BPW,), jnp.int32),
            pltpu.VMEM((_BPW, EMBED_DIM), jnp.float32),
            pltpu.VMEM((_BPW, EMBED_DIM), jnp.float32),
            pltpu.SemaphoreType.DMA,
            pltpu.SemaphoreType.DMA,
        ],
        compiler_params=pltpu.CompilerParams(use_tc_tiling_on_sc=False),
    )
    def emb_kernel(e1_hbm, rel_hbm, tab_e_hbm, tab_r_hbm, out_e_hbm,
                   out_r_hbm, idx_e, idx_r, rows_e, rows_r, sem_e, sem_r):
        wid = lax.axis_index("s") * _NC + lax.axis_index("c")
        base = wid * _BPW
        pltpu.sync_copy(e1_hbm.at[pl.ds(base, _BPW)], idx_e)
        pltpu.sync_copy(rel_hbm.at[pl.ds(base, _BPW)], idx_r)
        ce = pltpu.async_copy(tab_e_hbm.at[idx_e], rows_e, sem_e)
        cr = pltpu.async_copy(tab_r_hbm.at[idx_r], rows_r, sem_r)
        ce.wait()
        cr.wait()
        pltpu.sync_copy(rows_e, out_e_hbm.at[pl.ds(base, _BPW)])
        pltpu.sync_copy(rows_r, out_r_hbm.at[pl.ds(base, _BPW)])

    return emb_kernel


_emb_kernel = _make_kernel()


def kernel(e1, rel, emb_e_weight, emb_rel_weight):
    e1_flat = e1.reshape(BATCH)
    rel_flat = rel.reshape(BATCH)
    return _emb_kernel(e1_flat, rel_flat, emb_e_weight, emb_rel_weight)
